# baseline (speedup 1.0000x reference)
import functools

import jax
import jax.numpy as jnp
from jax import lax
from jax.experimental import pallas as pl
from jax.experimental.pallas import tpu as pltpu
from jax.experimental.pallas import tpu_sc as plsc


def kernel(x, table):
    b, h = x.shape
    _, d = table.shape
    idx = x.astype(jnp.int32)
    tbl_bits = lax.bitcast_convert_type(table, jnp.int32) ^ jnp.int32(1)

    info = plsc.get_sparse_core_info()
    nw = info.num_cores * info.num_subcores
    b_per_w = b // nw
    bchunk = 8
    nslot = 8
    steps = b_per_w // bchunk
    ngroups = steps // nslot
    assert b_per_w * nw == b and steps * bchunk == b_per_w
    assert ngroups * nslot == steps
    mesh = plsc.VectorSubcoreMesh(core_axis_name="c", subcore_axis_name="s")

    @functools.partial(
        pl.kernel,
        mesh=mesh,
        compiler_params=pltpu.CompilerParams(use_tc_tiling_on_sc=False),
        out_type=jax.ShapeDtypeStruct((b, h, d), jnp.int32),
        scratch_types=[
            pltpu.VMEM((b_per_w, h), jnp.int32),
            *[pltpu.VMEM((bchunk, h, d), jnp.int32) for _ in range(nslot)],
            *[pltpu.SemaphoreType.DMA for _ in range(2 * nslot)],
        ],
    )
    def emb(idx_hbm, table_hbm, out_hbm, idx_all, *refs):
        rows = refs[:nslot]
        gsem = refs[nslot:2 * nslot]
        ssem = refs[2 * nslot:]
        wid = lax.axis_index("s") * info.num_cores + lax.axis_index("c")
        bbase = wid * b_per_w
        pltpu.sync_copy(idx_hbm.at[pl.ds(bbase, b_per_w)], idx_all)

        def out_slice(i):
            return out_hbm.at[pl.ds(bbase + i * bchunk, bchunk)]

        def gather_start(i, s):
            for j in range(bchunk):
                pltpu.async_copy(
                    table_hbm.at[idx_all.at[i * bchunk + j]],
                    rows[s].at[j], gsem[s])

        def gather_wait(s):
            pltpu.make_async_copy(out_slice(0), rows[s], gsem[s]).wait()

        def store_wait(s):
            pltpu.make_async_copy(rows[s], out_slice(0), ssem[s]).wait()

        for s in range(nslot):
            gather_start(s, s)

        def body(g, carry):
            for s in range(nslot):
                i = g * nslot + s
                gather_wait(s)
                pltpu.async_copy(rows[s], out_slice(i), ssem[s])

            @pl.when(g + 1 < ngroups)
            def _():
                for s in range(nslot):
                    store_wait(s)
                    gather_start((g + 1) * nslot + s, s)

            return carry

        lax.fori_loop(0, ngroups, body, 0)
        for s in range(nslot):
            store_wait(s)

    out_bits = emb(idx, tbl_bits)
    return lax.bitcast_convert_type(out_bits ^ jnp.int32(1), jnp.float32)


# speedup vs baseline: 1.4324x; 1.4324x over previous
import functools

import jax
import jax.numpy as jnp
from jax import lax
from jax.experimental import pallas as pl
from jax.experimental.pallas import tpu as pltpu
from jax.experimental.pallas import tpu_sc as plsc


def kernel(x, table):
    b, h = x.shape
    v, d = table.shape
    idx = x.astype(jnp.int32)
    dp = 128
    tbl = jnp.pad(table, ((0, 0), (0, dp - d)))

    info = plsc.get_sparse_core_info()
    nw = info.num_cores * info.num_subcores
    b_per_w = b // nw
    bchunk = 2
    nslot = 8
    steps = b_per_w // bchunk
    ngroups = steps // nslot
    assert b_per_w * nw == b and steps * bchunk == b_per_w
    assert ngroups * nslot == steps
    mesh = plsc.VectorSubcoreMesh(core_axis_name="c", subcore_axis_name="s")

    @functools.partial(
        pl.kernel,
        mesh=mesh,
        compiler_params=pltpu.CompilerParams(use_tc_tiling_on_sc=False),
        out_type=jax.ShapeDtypeStruct((b, h, d), jnp.float32),
        scratch_types=[
            pltpu.VMEM((b_per_w, h), jnp.int32),
            *[pltpu.VMEM((bchunk, h, dp), jnp.float32) for _ in range(nslot)],
            *[pltpu.SemaphoreType.DMA for _ in range(2 * nslot)],
        ],
    )
    def emb(idx_hbm, table_hbm, out_hbm, idx_all, *refs):
        rows = refs[:nslot]
        gsem = refs[nslot:2 * nslot]
        ssem = refs[2 * nslot:]
        wid = lax.axis_index("s") * info.num_cores + lax.axis_index("c")
        bbase = wid * b_per_w
        pltpu.sync_copy(idx_hbm.at[pl.ds(bbase, b_per_w)], idx_all)

        def out_slice(i):
            return out_hbm.at[pl.ds(bbase + i * bchunk, bchunk)]

        def gather_start(i, s):
            for j in range(bchunk):
                pltpu.async_copy(
                    table_hbm.at[idx_all.at[i * bchunk + j]],
                    rows[s].at[j], gsem[s])

        def gather_wait(s):
            for j in range(bchunk):
                pltpu.make_async_copy(
                    table_hbm.at[idx_all.at[0]], rows[s].at[j],
                    gsem[s]).wait()

        def store_start(i, s):
            pltpu.async_copy(
                rows[s].at[:, :, pl.ds(0, d)], out_slice(i), ssem[s])

        def store_wait(s):
            pltpu.make_async_copy(
                rows[s].at[:, :, pl.ds(0, d)], out_slice(0), ssem[s]).wait()

        for s in range(nslot):
            gather_start(s, s)

        def body(g, carry):
            for s in range(nslot):
                i = g * nslot + s
                gather_wait(s)
                store_start(i, s)

            @pl.when(g + 1 < ngroups)
            def _():
                for s in range(nslot):
                    store_wait(s)
                    gather_start((g + 1) * nslot + s, s)

            return carry

        lax.fori_loop(0, ngroups, body, 0)
        for s in range(nslot):
            store_wait(s)

    return emb(idx, tbl)


# speedup vs baseline: 2.3490x; 1.6399x over previous
import functools

import jax
import jax.numpy as jnp
from jax import lax
from jax.experimental import pallas as pl
from jax.experimental.pallas import tpu as pltpu
from jax.experimental.pallas import tpu_sc as plsc


def kernel(x, table):
    b, h = x.shape
    _, d = table.shape
    idx = x.astype(jnp.int32)
    hp, dp = 56, 128

    info = plsc.get_sparse_core_info()
    nw = info.num_cores * info.num_subcores
    b_per_w = b // nw
    bchunk = 8
    nslot = 8
    steps = b_per_w // bchunk
    ngroups = steps // nslot
    assert b_per_w * nw == b and steps * bchunk == b_per_w
    assert ngroups * nslot == steps
    mesh = plsc.VectorSubcoreMesh(core_axis_name="c", subcore_axis_name="s")

    @functools.partial(
        pl.kernel,
        mesh=mesh,
        compiler_params=pltpu.CompilerParams(use_tc_tiling_on_sc=False),
        out_type=jax.ShapeDtypeStruct((b, hp, dp), jnp.float32),
        scratch_types=[
            pltpu.VMEM((b_per_w, h), jnp.int32),
            *[pltpu.VMEM((bchunk, h, d), jnp.float32) for _ in range(nslot)],
            *[pltpu.SemaphoreType.DMA for _ in range(2 * nslot)],
        ],
    )
    def emb(idx_hbm, table_hbm, out_hbm, idx_all, *refs):
        rows = refs[:nslot]
        gsem = refs[nslot:2 * nslot]
        ssem = refs[2 * nslot:]
        wid = lax.axis_index("s") * info.num_cores + lax.axis_index("c")
        bbase = wid * b_per_w
        pltpu.sync_copy(idx_hbm.at[pl.ds(bbase, b_per_w)], idx_all)

        def gather_start(i, s):
            for j in range(bchunk):
                pltpu.async_copy(
                    table_hbm.at[idx_all.at[i * bchunk + j]],
                    rows[s].at[j], gsem[s])

        def gather_wait(s):
            for j in range(bchunk):
                pltpu.make_async_copy(
                    table_hbm.at[idx_all.at[0]], rows[s].at[j],
                    gsem[s]).wait()

        def store_start(i, s):
            for j in range(bchunk):
                pltpu.async_copy(
                    rows[s].at[j],
                    out_hbm.at[bbase + i * bchunk + j,
                               pl.ds(0, h), pl.ds(0, d)],
                    ssem[s])

        def store_wait(s):
            for j in range(bchunk):
                pltpu.make_async_copy(
                    rows[s].at[j],
                    out_hbm.at[0, pl.ds(0, h), pl.ds(0, d)],
                    ssem[s]).wait()

        for s in range(nslot):
            gather_start(s, s)

        def body(g, carry):
            for s in range(nslot):
                i = g * nslot + s
                gather_wait(s)
                store_start(i, s)

            @pl.when(g + 1 < ngroups)
            def _():
                for s in range(nslot):
                    store_wait(s)
                    gather_start((g + 1) * nslot + s, s)

            return carry

        lax.fori_loop(0, ngroups, body, 0)
        for s in range(nslot):
            store_wait(s)

    out_padded = emb(idx, table)
    return out_padded[:, :h, :d]
